# baseline (speedup 1.0000x reference)
import functools

import jax
import jax.numpy as jnp
from jax.experimental import pallas as pl
from jax.experimental.pallas import tpu as pltpu

_K2 = 9
_TOP_K = 4
_EPS = 1e-8
_NC = 2


def _shift(x, di, dj, fill):
    h, w = x.shape[-2], x.shape[-1]
    if di > 0:
        pad = jnp.full(x.shape[:-2] + (di, w), fill, x.dtype)
        x = jnp.concatenate([x[..., di:, :], pad], axis=-2)
    elif di < 0:
        pad = jnp.full(x.shape[:-2] + (-di, w), fill, x.dtype)
        x = jnp.concatenate([pad, x[..., : h + di, :]], axis=-2)
    if dj > 0:
        pad = jnp.full(x.shape[:-2] + (h, dj), fill, x.dtype)
        x = jnp.concatenate([x[..., :, dj:], pad], axis=-1)
    elif dj < 0:
        pad = jnp.full(x.shape[:-2] + (h, -dj), fill, x.dtype)
        x = jnp.concatenate([pad, x[..., :, : w + dj]], axis=-1)
    return x


_FWD = ((0, 1), (1, -1), (1, 0), (1, 1))


def _accum_chunk(ref, dst):
    ch = ref[0]
    h, w = ch.shape[1], ch.shape[2]
    dst[0, :, :] = dst[0, :, :] + jnp.sum(ch * ch, axis=0)
    dst[1, :, : w - 1] = dst[1, :, : w - 1] + jnp.sum(
        ch[:, :, :-1] * ch[:, :, 1:], axis=0)
    dst[2, : h - 1, 1:] = dst[2, : h - 1, 1:] + jnp.sum(
        ch[:, :-1, 1:] * ch[:, 1:, :-1], axis=0)
    dst[3, : h - 1, :] = dst[3, : h - 1, :] + jnp.sum(
        ch[:, :-1, :] * ch[:, 1:, :], axis=0)
    dst[4, : h - 1, : w - 1] = dst[4, : h - 1, : w - 1] + jnp.sum(
        ch[:, :-1, :-1] * ch[:, 1:, 1:], axis=0)


def _sims_from_scratch(dst):
    norm2 = dst[0, :, :]
    n = jnp.sqrt(norm2)
    dn = jnp.maximum(n, _EPS)
    sims = [None] * _K2
    sims[4] = norm2 / (dn * dn)
    for k, (di, dj) in enumerate(_FWD):
        ns = jnp.maximum(_shift(n, di, dj, 0.0), _EPS)
        sims[3 * (di + 1) + (dj + 1)] = dst[k + 1, :, :] / (ns * dn)
    sims[3] = _shift(sims[5], 0, -1, 0.0)
    sims[1] = _shift(sims[7], -1, 0, 0.0)
    sims[0] = _shift(sims[8], -1, -1, 0.0)
    sims[2] = _shift(sims[6], -1, 1, 0.0)
    return sims


def _cross_prob_planes(logits_ref):
    lg = logits_ref[0]
    h, w = lg.shape[1], lg.shape[2]
    mx = jnp.max(lg, axis=0, keepdims=True)
    e = jnp.exp(lg - mx)
    prob = e / jnp.sum(e, axis=0, keepdims=True)
    u = jnp.float32(1.0 / lg.shape[0])
    row = jax.lax.broadcasted_iota(jnp.int32, (h, w), 0)
    col = jax.lax.broadcasted_iota(jnp.int32, (h, w), 1)
    cps = [None] * _K2
    cps[4] = jnp.sum(prob * prob, axis=0)
    rs = _shift(prob, 1, 0, 0.0)
    raw = {
        (0, 1): jnp.sum(prob * _shift(prob, 0, 1, 0.0), axis=0),
        (1, -1): jnp.sum(prob * _shift(rs, 0, -1, 0.0), axis=0),
        (1, 0): jnp.sum(prob * rs, axis=0),
        (1, 1): jnp.sum(prob * _shift(rs, 0, 1, 0.0), axis=0),
    }
    for (di, dj), v in raw.items():
        valid = jnp.ones((h, w), jnp.bool_)
        if di > 0:
            valid = valid & (row < h - di)
        if dj > 0:
            valid = valid & (col < w - dj)
        if dj < 0:
            valid = valid & (col >= -dj)
        cps[3 * (di + 1) + (dj + 1)] = jnp.where(valid, v, u)
    cps[3] = _shift(cps[5], 0, -1, u)
    cps[1] = _shift(cps[7], -1, 0, u)
    cps[0] = _shift(cps[8], -1, -1, u)
    cps[2] = _shift(cps[6], -1, 1, u)
    return cps


def _topk_sums(s, cp, mt):
    work = list(s)
    acc = jnp.zeros_like(s[0])
    for _ in range(_TOP_K + 1):
        m = functools.reduce(jnp.maximum, work)
        taken = jnp.zeros_like(m, jnp.bool_)
        val = jnp.zeros_like(m)
        for i in range(_K2):
            eq = (work[i] == m) & (~taken)
            val = jnp.where(eq, cp[i], val)
            work[i] = jnp.where(eq, jnp.float32(-3.0), work[i])
            taken = taken | eq
        acc = acc - m * val
    pos_sum = jnp.sum(acc * mt)

    work = list(s)
    acc = jnp.zeros_like(s[0])
    for _ in range(_TOP_K):
        m = functools.reduce(jnp.minimum, work)
        taken = jnp.zeros_like(m, jnp.bool_)
        val = jnp.zeros_like(m)
        for i in range(_K2):
            eq = (work[i] == m) & (~taken)
            val = jnp.where(eq, cp[i], val)
            work[i] = jnp.where(eq, jnp.float32(3.0), work[i])
            taken = taken | eq
        acc = acc - (1.0 - m) * (1.0 - val)
    neg_sum = jnp.sum(acc * mt)
    return pos_sum, neg_sum


def _kernel_body(logits_ref, gt_ref, ema_ref, src_ref, mm_ref, out_ref,
                 dsrc, dema, cps, acc_ref):
    b = pl.program_id(0)
    c = pl.program_id(1)
    h, w = gt_ref.shape[2], gt_ref.shape[3]

    @pl.when(c == 0)
    def _init():
        z = jnp.zeros((5, h, w), jnp.float32)
        dsrc[:, :, :] = z
        dema[:, :, :] = z
        planes = _cross_prob_planes(logits_ref)
        for i in range(_K2):
            cps[i, :, :] = planes[i]

    _accum_chunk(src_ref, dsrc)
    _accum_chunk(ema_ref, dema)

    @pl.when(c == _NC - 1)
    def _tail():
        g = gt_ref[0, 0]
        ig = (g != 255).astype(jnp.float32)
        src_sims = _sims_from_scratch(dsrc)
        sps = jnp.float32(0.0)
        spc = jnp.float32(0.0)
        sns = jnp.float32(0.0)
        snc = jnp.float32(0.0)
        for idx in range(_K2):
            di, dj = idx // 3 - 1, idx % 3 - 1
            pos = (_shift(g, di, dj, 0) == g).astype(jnp.float32)
            mp = pos * ig
            mn = (1.0 - pos) * ig
            sps = sps + jnp.sum(src_sims[idx] * mp)
            spc = spc + jnp.sum(mp)
            sns = sns + jnp.sum(src_sims[idx] * mn)
            snc = snc + jnp.sum(mn)

        s = _sims_from_scratch(dema)
        cp = [cps[i, :, :] for i in range(_K2)]
        mt = ((1.0 - mm_ref[0, 0]) > 0.5).astype(jnp.float32)
        tc = jnp.sum(mt)
        lps, lns = _topk_sums(s, cp, mt)

        part = jnp.concatenate(
            [v.reshape(1, 1) for v in
             (sps, spc, sns, snc, lps, lns, tc, jnp.float32(0.0))], axis=1)
        new = jnp.where(b == 0, part, acc_ref[:, :] + part)
        acc_ref[:, :] = new

        src_pos_mean = new[0, 0] / jnp.maximum(new[0, 1], 1.0)
        src_neg_mean = new[0, 2] / jnp.maximum(new[0, 3], 1.0)
        loss_sim_pos = new[0, 4] / jnp.maximum((_TOP_K + 1) * new[0, 6], 1.0)
        loss_sim_neg = new[0, 5] / jnp.maximum(_TOP_K * new[0, 6], 1.0)
        out_ref[:, :] = jnp.concatenate(
            [
                (-src_pos_mean).reshape(1, 1),
                src_neg_mean.reshape(1, 1),
                loss_sim_pos.reshape(1, 1),
                loss_sim_neg.reshape(1, 1),
            ],
            axis=1,
        )


def kernel(logits_trg, gt_src, x_ema, x_src, img_trg, mix_masks):
    del img_trg
    B, C, H, W = logits_trg.shape
    Cf = x_ema.shape[1]
    ck = Cf // _NC
    gt = gt_src.astype(jnp.int32)
    out = pl.pallas_call(
        _kernel_body,
        grid=(B, _NC),
        in_specs=[
            pl.BlockSpec((1, C, H, W), lambda b, c: (b, 0, 0, 0)),
            pl.BlockSpec((1, 1, H, W), lambda b, c: (b, 0, 0, 0)),
            pl.BlockSpec((1, ck, H, W), lambda b, c: (b, c, 0, 0)),
            pl.BlockSpec((1, ck, H, W), lambda b, c: (b, c, 0, 0)),
            pl.BlockSpec((1, 1, H, W), lambda b, c: (b, 0, 0, 0)),
        ],
        out_specs=pl.BlockSpec((1, 4), lambda b, c: (0, 0)),
        out_shape=jax.ShapeDtypeStruct((1, 4), jnp.float32),
        scratch_shapes=[
            pltpu.VMEM((5, H, W), jnp.float32),
            pltpu.VMEM((5, H, W), jnp.float32),
            pltpu.VMEM((_K2, H, W), jnp.float32),
            pltpu.VMEM((1, 8), jnp.float32),
        ],
        compiler_params=pltpu.CompilerParams(
            dimension_semantics=("arbitrary", "arbitrary")),
    )(logits_trg, gt, x_ema, x_src, mix_masks)
    return out[0]


# speedup vs baseline: 1.0278x; 1.0278x over previous
import functools

import jax
import jax.numpy as jnp
from jax.experimental import pallas as pl
from jax.experimental.pallas import tpu as pltpu

_K2 = 9
_TOP_K = 4
_EPS = 1e-8
_NC = 2


def _shift(x, di, dj, fill):
    h, w = x.shape[-2], x.shape[-1]
    if di > 0:
        pad = jnp.full(x.shape[:-2] + (di, w), fill, x.dtype)
        x = jnp.concatenate([x[..., di:, :], pad], axis=-2)
    elif di < 0:
        pad = jnp.full(x.shape[:-2] + (-di, w), fill, x.dtype)
        x = jnp.concatenate([pad, x[..., : h + di, :]], axis=-2)
    if dj > 0:
        pad = jnp.full(x.shape[:-2] + (h, dj), fill, x.dtype)
        x = jnp.concatenate([x[..., :, dj:], pad], axis=-1)
    elif dj < 0:
        pad = jnp.full(x.shape[:-2] + (h, -dj), fill, x.dtype)
        x = jnp.concatenate([pad, x[..., :, : w + dj]], axis=-1)
    return x


_FWD = ((0, 1), (1, -1), (1, 0), (1, 1))


def _accum_chunk(ref, dst):
    ch = ref[0]
    h = ch.shape[1]
    lsh = _shift(ch, 0, 1, 0.0)
    dst[0, :, :] = dst[0, :, :] + jnp.sum(ch * ch, axis=0)
    dst[1, :, :] = dst[1, :, :] + jnp.sum(ch * lsh, axis=0)
    dst[2, : h - 1, :] = dst[2, : h - 1, :] + jnp.sum(
        ch[:, 1:, :] * lsh[:, :-1, :], axis=0)
    dst[3, : h - 1, :] = dst[3, : h - 1, :] + jnp.sum(
        ch[:, :-1, :] * ch[:, 1:, :], axis=0)
    dst[4, : h - 1, :] = dst[4, : h - 1, :] + jnp.sum(
        ch[:, :-1, :] * lsh[:, 1:, :], axis=0)


def _sims_from_scratch(dst):
    norm2 = dst[0, :, :]
    n = jnp.sqrt(norm2)
    dn = jnp.maximum(n, _EPS)
    sims = [None] * _K2
    sims[4] = norm2 / (dn * dn)
    dots = (dst[1, :, :], _shift(dst[2, :, :], 0, -1, 0.0),
            dst[3, :, :], dst[4, :, :])
    for (di, dj), dot in zip(_FWD, dots):
        ns = jnp.maximum(_shift(n, di, dj, 0.0), _EPS)
        sims[3 * (di + 1) + (dj + 1)] = dot / (ns * dn)
    sims[3] = _shift(sims[5], 0, -1, 0.0)
    sims[1] = _shift(sims[7], -1, 0, 0.0)
    sims[0] = _shift(sims[8], -1, -1, 0.0)
    sims[2] = _shift(sims[6], -1, 1, 0.0)
    return sims


def _cross_prob_planes(logits_ref):
    lg = logits_ref[0]
    h, w = lg.shape[1], lg.shape[2]
    mx = jnp.max(lg, axis=0, keepdims=True)
    e = jnp.exp(lg - mx)
    prob = e / jnp.sum(e, axis=0, keepdims=True)
    u = jnp.float32(1.0 / lg.shape[0])
    row = jax.lax.broadcasted_iota(jnp.int32, (h, w), 0)
    col = jax.lax.broadcasted_iota(jnp.int32, (h, w), 1)
    cps = [None] * _K2
    cps[4] = jnp.sum(prob * prob, axis=0)
    rs = _shift(prob, 1, 0, 0.0)
    raw = {
        (0, 1): jnp.sum(prob * _shift(prob, 0, 1, 0.0), axis=0),
        (1, -1): jnp.sum(prob * _shift(rs, 0, -1, 0.0), axis=0),
        (1, 0): jnp.sum(prob * rs, axis=0),
        (1, 1): jnp.sum(prob * _shift(rs, 0, 1, 0.0), axis=0),
    }
    for (di, dj), v in raw.items():
        valid = jnp.ones((h, w), jnp.bool_)
        if di > 0:
            valid = valid & (row < h - di)
        if dj > 0:
            valid = valid & (col < w - dj)
        if dj < 0:
            valid = valid & (col >= -dj)
        cps[3 * (di + 1) + (dj + 1)] = jnp.where(valid, v, u)
    cps[3] = _shift(cps[5], 0, -1, u)
    cps[1] = _shift(cps[7], -1, 0, u)
    cps[0] = _shift(cps[8], -1, -1, u)
    cps[2] = _shift(cps[6], -1, 1, u)
    return cps


def _topk_sums(s, cp, mt):
    work = list(s)
    acc = jnp.zeros_like(s[0])
    for _ in range(_TOP_K + 1):
        m = functools.reduce(jnp.maximum, work)
        taken = jnp.zeros_like(m, jnp.bool_)
        val = jnp.zeros_like(m)
        for i in range(_K2):
            eq = (work[i] == m) & (~taken)
            val = jnp.where(eq, cp[i], val)
            work[i] = jnp.where(eq, jnp.float32(-3.0), work[i])
            taken = taken | eq
        acc = acc - m * val
    pos_sum = jnp.sum(acc * mt)

    work = list(s)
    acc = jnp.zeros_like(s[0])
    for _ in range(_TOP_K):
        m = functools.reduce(jnp.minimum, work)
        taken = jnp.zeros_like(m, jnp.bool_)
        val = jnp.zeros_like(m)
        for i in range(_K2):
            eq = (work[i] == m) & (~taken)
            val = jnp.where(eq, cp[i], val)
            work[i] = jnp.where(eq, jnp.float32(3.0), work[i])
            taken = taken | eq
        acc = acc - (1.0 - m) * (1.0 - val)
    neg_sum = jnp.sum(acc * mt)
    return pos_sum, neg_sum


def _kernel_body(logits_ref, gt_ref, ema_ref, src_ref, mm_ref, out_ref,
                 dsrc, dema, cps, acc_ref):
    b = pl.program_id(0)
    c = pl.program_id(1)
    h, w = gt_ref.shape[2], gt_ref.shape[3]

    @pl.when(c == 0)
    def _init():
        z = jnp.zeros((5, h, w), jnp.float32)
        dsrc[:, :, :] = z
        dema[:, :, :] = z
        planes = _cross_prob_planes(logits_ref)
        for i in range(_K2):
            cps[i, :, :] = planes[i]

    _accum_chunk(src_ref, dsrc)
    _accum_chunk(ema_ref, dema)

    @pl.when(c == _NC - 1)
    def _tail():
        g = gt_ref[0, 0]
        ig = (g != 255).astype(jnp.float32)
        src_sims = _sims_from_scratch(dsrc)
        sps = jnp.float32(0.0)
        spc = jnp.float32(0.0)
        stot = jnp.float32(0.0)
        for idx in range(_K2):
            di, dj = idx // 3 - 1, idx % 3 - 1
            pos = (_shift(g, di, dj, 0) == g).astype(jnp.float32)
            mp = pos * ig
            sps = sps + jnp.sum(src_sims[idx] * mp)
            spc = spc + jnp.sum(mp)
            stot = stot + jnp.sum(src_sims[idx] * ig)
        sns = stot - sps
        snc = _K2 * jnp.sum(ig) - spc

        s = _sims_from_scratch(dema)
        cp = [cps[i, :, :] for i in range(_K2)]
        mt = ((1.0 - mm_ref[0, 0]) > 0.5).astype(jnp.float32)
        tc = jnp.sum(mt)
        lps, lns = _topk_sums(s, cp, mt)

        part = jnp.concatenate(
            [v.reshape(1, 1) for v in
             (sps, spc, sns, snc, lps, lns, tc, jnp.float32(0.0))], axis=1)
        new = jnp.where(b == 0, part, acc_ref[:, :] + part)
        acc_ref[:, :] = new

        src_pos_mean = new[0, 0] / jnp.maximum(new[0, 1], 1.0)
        src_neg_mean = new[0, 2] / jnp.maximum(new[0, 3], 1.0)
        loss_sim_pos = new[0, 4] / jnp.maximum((_TOP_K + 1) * new[0, 6], 1.0)
        loss_sim_neg = new[0, 5] / jnp.maximum(_TOP_K * new[0, 6], 1.0)
        out_ref[:, :] = jnp.concatenate(
            [
                (-src_pos_mean).reshape(1, 1),
                src_neg_mean.reshape(1, 1),
                loss_sim_pos.reshape(1, 1),
                loss_sim_neg.reshape(1, 1),
            ],
            axis=1,
        )


def kernel(logits_trg, gt_src, x_ema, x_src, img_trg, mix_masks):
    del img_trg
    B, C, H, W = logits_trg.shape
    Cf = x_ema.shape[1]
    ck = Cf // _NC
    gt = gt_src.astype(jnp.int32)
    out = pl.pallas_call(
        _kernel_body,
        grid=(B, _NC),
        in_specs=[
            pl.BlockSpec((1, C, H, W), lambda b, c: (b, 0, 0, 0)),
            pl.BlockSpec((1, 1, H, W), lambda b, c: (b, 0, 0, 0)),
            pl.BlockSpec((1, ck, H, W), lambda b, c: (b, c, 0, 0)),
            pl.BlockSpec((1, ck, H, W), lambda b, c: (b, c, 0, 0)),
            pl.BlockSpec((1, 1, H, W), lambda b, c: (b, 0, 0, 0)),
        ],
        out_specs=pl.BlockSpec((1, 4), lambda b, c: (0, 0)),
        out_shape=jax.ShapeDtypeStruct((1, 4), jnp.float32),
        scratch_shapes=[
            pltpu.VMEM((5, H, W), jnp.float32),
            pltpu.VMEM((5, H, W), jnp.float32),
            pltpu.VMEM((_K2, H, W), jnp.float32),
            pltpu.VMEM((1, 8), jnp.float32),
        ],
        compiler_params=pltpu.CompilerParams(
            dimension_semantics=("arbitrary", "arbitrary")),
    )(logits_trg, gt, x_ema, x_src, mix_masks)
    return out[0]


# speedup vs baseline: 1.0755x; 1.0465x over previous
import jax
import jax.numpy as jnp
from jax.experimental import pallas as pl
from jax.experimental.pallas import tpu as pltpu

_K2 = 9
_TOP_K = 4
_EPS = 1e-8
_NC = 2


def _shift(x, di, dj, fill):
    h, w = x.shape[-2], x.shape[-1]
    if di > 0:
        pad = jnp.full(x.shape[:-2] + (di, w), fill, x.dtype)
        x = jnp.concatenate([x[..., di:, :], pad], axis=-2)
    elif di < 0:
        pad = jnp.full(x.shape[:-2] + (-di, w), fill, x.dtype)
        x = jnp.concatenate([pad, x[..., : h + di, :]], axis=-2)
    if dj > 0:
        pad = jnp.full(x.shape[:-2] + (h, dj), fill, x.dtype)
        x = jnp.concatenate([x[..., :, dj:], pad], axis=-1)
    elif dj < 0:
        pad = jnp.full(x.shape[:-2] + (h, -dj), fill, x.dtype)
        x = jnp.concatenate([pad, x[..., :, : w + dj]], axis=-1)
    return x


_FWD = ((0, 1), (1, -1), (1, 0), (1, 1))


def _accum_chunk(ref, dst):
    ch = ref[0]
    h = ch.shape[1]
    lsh = _shift(ch, 0, 1, 0.0)
    dst[0, :, :] = dst[0, :, :] + jnp.sum(ch * ch, axis=0)
    dst[1, :, :] = dst[1, :, :] + jnp.sum(ch * lsh, axis=0)
    dst[2, : h - 1, :] = dst[2, : h - 1, :] + jnp.sum(
        ch[:, 1:, :] * lsh[:, :-1, :], axis=0)
    dst[3, : h - 1, :] = dst[3, : h - 1, :] + jnp.sum(
        ch[:, :-1, :] * ch[:, 1:, :], axis=0)
    dst[4, : h - 1, :] = dst[4, : h - 1, :] + jnp.sum(
        ch[:, :-1, :] * lsh[:, 1:, :], axis=0)


def _sims_from_scratch(dst):
    norm2 = dst[0, :, :]
    n = jnp.sqrt(norm2)
    dn = jnp.maximum(n, _EPS)
    sims = [None] * _K2
    sims[4] = norm2 / (dn * dn)
    dots = (dst[1, :, :], _shift(dst[2, :, :], 0, -1, 0.0),
            dst[3, :, :], dst[4, :, :])
    for (di, dj), dot in zip(_FWD, dots):
        ns = jnp.maximum(_shift(n, di, dj, 0.0), _EPS)
        sims[3 * (di + 1) + (dj + 1)] = dot / (ns * dn)
    sims[3] = _shift(sims[5], 0, -1, 0.0)
    sims[1] = _shift(sims[7], -1, 0, 0.0)
    sims[0] = _shift(sims[8], -1, -1, 0.0)
    sims[2] = _shift(sims[6], -1, 1, 0.0)
    return sims


def _cross_prob_planes(logits_ref):
    lg = logits_ref[0]
    h, w = lg.shape[1], lg.shape[2]
    mx = jnp.max(lg, axis=0, keepdims=True)
    e = jnp.exp(lg - mx)
    prob = e / jnp.sum(e, axis=0, keepdims=True)
    u = jnp.float32(1.0 / lg.shape[0])
    row = jax.lax.broadcasted_iota(jnp.int32, (h, w), 0)
    col = jax.lax.broadcasted_iota(jnp.int32, (h, w), 1)
    cps = [None] * _K2
    cps[4] = jnp.sum(prob * prob, axis=0)
    rs = _shift(prob, 1, 0, 0.0)
    raw = {
        (0, 1): jnp.sum(prob * _shift(prob, 0, 1, 0.0), axis=0),
        (1, -1): jnp.sum(prob * _shift(rs, 0, -1, 0.0), axis=0),
        (1, 0): jnp.sum(prob * rs, axis=0),
        (1, 1): jnp.sum(prob * _shift(rs, 0, 1, 0.0), axis=0),
    }
    for (di, dj), v in raw.items():
        valid = jnp.ones((h, w), jnp.bool_)
        if di > 0:
            valid = valid & (row < h - di)
        if dj > 0:
            valid = valid & (col < w - dj)
        if dj < 0:
            valid = valid & (col >= -dj)
        cps[3 * (di + 1) + (dj + 1)] = jnp.where(valid, v, u)
    cps[3] = _shift(cps[5], 0, -1, u)
    cps[1] = _shift(cps[7], -1, 0, u)
    cps[0] = _shift(cps[8], -1, -1, u)
    cps[2] = _shift(cps[6], -1, 1, u)
    return cps


def _topk_sums(s, cp, mt):
    rank = [jnp.full(s[0].shape, float(i), jnp.float32) for i in range(_K2)]
    for i in range(_K2):
        for j in range(i + 1, _K2):
            f = (s[i] < s[j]).astype(jnp.float32)
            rank[i] = rank[i] + f
            rank[j] = rank[j] - f
    pos = jnp.zeros_like(s[0])
    neg_all = jnp.zeros_like(s[0])
    neg_sel = jnp.zeros_like(s[0])
    for i in range(_K2):
        sel = rank[i] < (_TOP_K + 0.5)
        t1 = s[i] * cp[i]
        t2 = (1.0 - s[i]) * (1.0 - cp[i])
        pos = pos + jnp.where(sel, t1, 0.0)
        neg_all = neg_all + t2
        neg_sel = neg_sel + jnp.where(sel, t2, 0.0)
    pos_sum = -jnp.sum(pos * mt)
    neg_sum = -jnp.sum((neg_all - neg_sel) * mt)
    return pos_sum, neg_sum


def _kernel_body(logits_ref, gt_ref, ema_ref, src_ref, mm_ref, out_ref,
                 dsrc, dema, cps, acc_ref):
    b = pl.program_id(0)
    c = pl.program_id(1)
    h, w = gt_ref.shape[2], gt_ref.shape[3]

    @pl.when(c == 0)
    def _init():
        z = jnp.zeros((5, h, w), jnp.float32)
        dsrc[:, :, :] = z
        dema[:, :, :] = z
        planes = _cross_prob_planes(logits_ref)
        for i in range(_K2):
            cps[i, :, :] = planes[i]

    _accum_chunk(src_ref, dsrc)
    _accum_chunk(ema_ref, dema)

    @pl.when(c == _NC - 1)
    def _tail():
        g = gt_ref[0, 0]
        ig = (g != 255).astype(jnp.float32)
        src_sims = _sims_from_scratch(dsrc)
        sps = jnp.float32(0.0)
        spc = jnp.float32(0.0)
        stot = jnp.float32(0.0)
        for idx in range(_K2):
            di, dj = idx // 3 - 1, idx % 3 - 1
            pos = (_shift(g, di, dj, 0) == g).astype(jnp.float32)
            mp = pos * ig
            sps = sps + jnp.sum(src_sims[idx] * mp)
            spc = spc + jnp.sum(mp)
            stot = stot + jnp.sum(src_sims[idx] * ig)
        sns = stot - sps
        snc = _K2 * jnp.sum(ig) - spc

        s = _sims_from_scratch(dema)
        cp = [cps[i, :, :] for i in range(_K2)]
        mt = ((1.0 - mm_ref[0, 0]) > 0.5).astype(jnp.float32)
        tc = jnp.sum(mt)
        lps, lns = _topk_sums(s, cp, mt)

        part = jnp.concatenate(
            [v.reshape(1, 1) for v in
             (sps, spc, sns, snc, lps, lns, tc, jnp.float32(0.0))], axis=1)
        new = jnp.where(b == 0, part, acc_ref[:, :] + part)
        acc_ref[:, :] = new

        src_pos_mean = new[0, 0] / jnp.maximum(new[0, 1], 1.0)
        src_neg_mean = new[0, 2] / jnp.maximum(new[0, 3], 1.0)
        loss_sim_pos = new[0, 4] / jnp.maximum((_TOP_K + 1) * new[0, 6], 1.0)
        loss_sim_neg = new[0, 5] / jnp.maximum(_TOP_K * new[0, 6], 1.0)
        out_ref[:, :] = jnp.concatenate(
            [
                (-src_pos_mean).reshape(1, 1),
                src_neg_mean.reshape(1, 1),
                loss_sim_pos.reshape(1, 1),
                loss_sim_neg.reshape(1, 1),
            ],
            axis=1,
        )


def kernel(logits_trg, gt_src, x_ema, x_src, img_trg, mix_masks):
    del img_trg
    B, C, H, W = logits_trg.shape
    Cf = x_ema.shape[1]
    ck = Cf // _NC
    gt = gt_src.astype(jnp.int32)
    out = pl.pallas_call(
        _kernel_body,
        grid=(B, _NC),
        in_specs=[
            pl.BlockSpec((1, C, H, W), lambda b, c: (b, 0, 0, 0)),
            pl.BlockSpec((1, 1, H, W), lambda b, c: (b, 0, 0, 0)),
            pl.BlockSpec((1, ck, H, W), lambda b, c: (b, c, 0, 0)),
            pl.BlockSpec((1, ck, H, W), lambda b, c: (b, c, 0, 0)),
            pl.BlockSpec((1, 1, H, W), lambda b, c: (b, 0, 0, 0)),
        ],
        out_specs=pl.BlockSpec((1, 4), lambda b, c: (0, 0)),
        out_shape=jax.ShapeDtypeStruct((1, 4), jnp.float32),
        scratch_shapes=[
            pltpu.VMEM((5, H, W), jnp.float32),
            pltpu.VMEM((5, H, W), jnp.float32),
            pltpu.VMEM((_K2, H, W), jnp.float32),
            pltpu.VMEM((1, 8), jnp.float32),
        ],
        compiler_params=pltpu.CompilerParams(
            dimension_semantics=("arbitrary", "arbitrary")),
    )(logits_trg, gt, x_ema, x_src, mix_masks)
    return out[0]


# speedup vs baseline: 1.0767x; 1.0012x over previous
import jax
import jax.numpy as jnp
from jax.experimental import pallas as pl
from jax.experimental.pallas import tpu as pltpu

_K2 = 9
_TOP_K = 4
_EPS = 1e-8
_NC = 2


def _shift(x, di, dj, fill):
    h, w = x.shape[-2], x.shape[-1]
    if di > 0:
        pad = jnp.full(x.shape[:-2] + (di, w), fill, x.dtype)
        x = jnp.concatenate([x[..., di:, :], pad], axis=-2)
    elif di < 0:
        pad = jnp.full(x.shape[:-2] + (-di, w), fill, x.dtype)
        x = jnp.concatenate([pad, x[..., : h + di, :]], axis=-2)
    if dj > 0:
        pad = jnp.full(x.shape[:-2] + (h, dj), fill, x.dtype)
        x = jnp.concatenate([x[..., :, dj:], pad], axis=-1)
    elif dj < 0:
        pad = jnp.full(x.shape[:-2] + (h, -dj), fill, x.dtype)
        x = jnp.concatenate([pad, x[..., :, : w + dj]], axis=-1)
    return x


_FWD = ((0, 1), (1, -1), (1, 0), (1, 1))


def _accum_chunk(ref, dst):
    ch = ref[0]
    h = ch.shape[1]
    lsh = _shift(ch, 0, 1, 0.0)
    dst[0, :, :] = dst[0, :, :] + jnp.sum(ch * ch, axis=0)
    dst[1, :, :] = dst[1, :, :] + jnp.sum(ch * lsh, axis=0)
    dst[2, : h - 1, :] = dst[2, : h - 1, :] + jnp.sum(
        ch[:, 1:, :] * lsh[:, :-1, :], axis=0)
    dst[3, : h - 1, :] = dst[3, : h - 1, :] + jnp.sum(
        ch[:, :-1, :] * ch[:, 1:, :], axis=0)
    dst[4, : h - 1, :] = dst[4, : h - 1, :] + jnp.sum(
        ch[:, :-1, :] * lsh[:, 1:, :], axis=0)


def _sims_from_scratch(dst):
    norm2 = dst[0, :, :]
    n = jnp.sqrt(norm2)
    inv = 1.0 / jnp.maximum(n, _EPS)
    fill = float(1.0 / _EPS)
    sims = [None] * _K2
    sims[4] = norm2 * (inv * inv)
    dots = (dst[1, :, :], _shift(dst[2, :, :], 0, -1, 0.0),
            dst[3, :, :], dst[4, :, :])
    for (di, dj), dot in zip(_FWD, dots):
        sims[3 * (di + 1) + (dj + 1)] = dot * (_shift(inv, di, dj, fill) * inv)
    sims[3] = _shift(sims[5], 0, -1, 0.0)
    sims[1] = _shift(sims[7], -1, 0, 0.0)
    sims[0] = _shift(sims[8], -1, -1, 0.0)
    sims[2] = _shift(sims[6], -1, 1, 0.0)
    return sims


def _cross_prob_planes(logits_ref):
    lg = logits_ref[0]
    h, w = lg.shape[1], lg.shape[2]
    mx = jnp.max(lg, axis=0, keepdims=True)
    e = jnp.exp(lg - mx)
    prob = e / jnp.sum(e, axis=0, keepdims=True)
    u = jnp.float32(1.0 / lg.shape[0])
    row = jax.lax.broadcasted_iota(jnp.int32, (h, w), 0)
    col = jax.lax.broadcasted_iota(jnp.int32, (h, w), 1)
    cps = [None] * _K2
    cps[4] = jnp.sum(prob * prob, axis=0)
    rs = _shift(prob, 1, 0, 0.0)
    raw = {
        (0, 1): jnp.sum(prob * _shift(prob, 0, 1, 0.0), axis=0),
        (1, -1): jnp.sum(prob * _shift(rs, 0, -1, 0.0), axis=0),
        (1, 0): jnp.sum(prob * rs, axis=0),
        (1, 1): jnp.sum(prob * _shift(rs, 0, 1, 0.0), axis=0),
    }
    for (di, dj), v in raw.items():
        valid = jnp.ones((h, w), jnp.bool_)
        if di > 0:
            valid = valid & (row < h - di)
        if dj > 0:
            valid = valid & (col < w - dj)
        if dj < 0:
            valid = valid & (col >= -dj)
        cps[3 * (di + 1) + (dj + 1)] = jnp.where(valid, v, u)
    cps[3] = _shift(cps[5], 0, -1, u)
    cps[1] = _shift(cps[7], -1, 0, u)
    cps[0] = _shift(cps[8], -1, -1, u)
    cps[2] = _shift(cps[6], -1, 1, u)
    return cps


def _topk_sums(s, cp, mt):
    rank = [jnp.full(s[0].shape, float(i), jnp.float32) for i in range(_K2)]
    for i in range(_K2):
        for j in range(i + 1, _K2):
            f = (s[i] < s[j]).astype(jnp.float32)
            rank[i] = rank[i] + f
            rank[j] = rank[j] - f
    pos = jnp.zeros_like(s[0])
    neg_all = jnp.zeros_like(s[0])
    neg_sel = jnp.zeros_like(s[0])
    for i in range(_K2):
        sel = rank[i] < (_TOP_K + 0.5)
        t1 = s[i] * cp[i]
        t2 = (1.0 - s[i]) * (1.0 - cp[i])
        pos = pos + jnp.where(sel, t1, 0.0)
        neg_all = neg_all + t2
        neg_sel = neg_sel + jnp.where(sel, t2, 0.0)
    pos_sum = -jnp.sum(pos * mt)
    neg_sum = -jnp.sum((neg_all - neg_sel) * mt)
    return pos_sum, neg_sum


def _kernel_body(logits_ref, gt_ref, ema_ref, src_ref, mm_ref, out_ref,
                 dsrc, dema, cps, acc_ref):
    b = pl.program_id(0)
    c = pl.program_id(1)
    h, w = gt_ref.shape[2], gt_ref.shape[3]

    @pl.when(c == 0)
    def _init():
        z = jnp.zeros((5, h, w), jnp.float32)
        dsrc[:, :, :] = z
        dema[:, :, :] = z
        planes = _cross_prob_planes(logits_ref)
        for i in range(_K2):
            cps[i, :, :] = planes[i]

    _accum_chunk(src_ref, dsrc)
    _accum_chunk(ema_ref, dema)

    @pl.when(c == _NC - 1)
    def _tail():
        g = gt_ref[0, 0]
        ig = (g != 255).astype(jnp.float32)
        src_sims = _sims_from_scratch(dsrc)
        sps_p = jnp.zeros((h, w), jnp.float32)
        spc_p = jnp.zeros((h, w), jnp.float32)
        stot_p = jnp.zeros((h, w), jnp.float32)
        for idx in range(_K2):
            di, dj = idx // 3 - 1, idx % 3 - 1
            mp = jnp.where(_shift(g, di, dj, 0) == g, ig, 0.0)
            sps_p = sps_p + src_sims[idx] * mp
            spc_p = spc_p + mp
            stot_p = stot_p + src_sims[idx]
        sps = jnp.sum(sps_p)
        spc = jnp.sum(spc_p)
        sns = jnp.sum(stot_p * ig) - sps
        snc = _K2 * jnp.sum(ig) - spc

        s = _sims_from_scratch(dema)
        cp = [cps[i, :, :] for i in range(_K2)]
        mt = ((1.0 - mm_ref[0, 0]) > 0.5).astype(jnp.float32)
        tc = jnp.sum(mt)
        lps, lns = _topk_sums(s, cp, mt)

        part = jnp.concatenate(
            [v.reshape(1, 1) for v in
             (sps, spc, sns, snc, lps, lns, tc, jnp.float32(0.0))], axis=1)
        new = jnp.where(b == 0, part, acc_ref[:, :] + part)
        acc_ref[:, :] = new

        src_pos_mean = new[0, 0] / jnp.maximum(new[0, 1], 1.0)
        src_neg_mean = new[0, 2] / jnp.maximum(new[0, 3], 1.0)
        loss_sim_pos = new[0, 4] / jnp.maximum((_TOP_K + 1) * new[0, 6], 1.0)
        loss_sim_neg = new[0, 5] / jnp.maximum(_TOP_K * new[0, 6], 1.0)
        out_ref[:, :] = jnp.concatenate(
            [
                (-src_pos_mean).reshape(1, 1),
                src_neg_mean.reshape(1, 1),
                loss_sim_pos.reshape(1, 1),
                loss_sim_neg.reshape(1, 1),
            ],
            axis=1,
        )


def kernel(logits_trg, gt_src, x_ema, x_src, img_trg, mix_masks):
    del img_trg
    B, C, H, W = logits_trg.shape
    Cf = x_ema.shape[1]
    ck = Cf // _NC
    gt = gt_src.astype(jnp.int32)
    out = pl.pallas_call(
        _kernel_body,
        grid=(B, _NC),
        in_specs=[
            pl.BlockSpec((1, C, H, W), lambda b, c: (b, 0, 0, 0)),
            pl.BlockSpec((1, 1, H, W), lambda b, c: (b, 0, 0, 0)),
            pl.BlockSpec((1, ck, H, W), lambda b, c: (b, c, 0, 0)),
            pl.BlockSpec((1, ck, H, W), lambda b, c: (b, c, 0, 0)),
            pl.BlockSpec((1, 1, H, W), lambda b, c: (b, 0, 0, 0)),
        ],
        out_specs=pl.BlockSpec((1, 4), lambda b, c: (0, 0)),
        out_shape=jax.ShapeDtypeStruct((1, 4), jnp.float32),
        scratch_shapes=[
            pltpu.VMEM((5, H, W), jnp.float32),
            pltpu.VMEM((5, H, W), jnp.float32),
            pltpu.VMEM((_K2, H, W), jnp.float32),
            pltpu.VMEM((1, 8), jnp.float32),
        ],
        compiler_params=pltpu.CompilerParams(
            dimension_semantics=("arbitrary", "arbitrary")),
    )(logits_trg, gt, x_ema, x_src, mix_masks)
    return out[0]
